# baseline (speedup 1.0000x reference)
import jax
import jax.numpy as jnp
from jax import lax
from jax.experimental import pallas as pl
from jax.experimental.pallas import tpu as pltpu, tpu_sc as plsc

_B = 16384
_D = 64
_NC, _NS = 2, 16
_L = 16
_SEG = 1024
_CHW = 256
_GB = 4
_JBITS = 14
_HB = _B // 2
_SROWS = _HB + _L

_NU = 1000000
_NM = 100000
_U_MAIN = 999936
_U_CK_SC = 1953
_U_SPLIT = 499968
_M_MAIN = 99840
_M_CK_SC = 195
_M_SPLIT = 49920


def _stream_table(idx_hbm, tbl_hbm, out01, ck_base, nck_sc,
                  idxv, hits, bufa, bufb, stage, jbuf, shared,
                  sema, semb, semi, semsc, cid, sid):
    start = ck_base + sid * nck_sc // _NS
    end = ck_base + (sid + 1) * nck_sc // _NS
    nck = end - start
    c_lo = start * _CHW
    c_hi = end * _CHW
    iota = lax.iota(jnp.int32, _L)

    def seg_step(s, offv):
        pltpu.make_async_copy(
            idx_hbm.at[pl.ds(s * _SEG, _SEG)], idxv, semi).wait()

        def scan_step(t, offv):
            v = idxv[pl.ds(t * _L, _L)]
            jv = iota + (s * _SEG + t * _L)
            m = (v >= c_lo) & (v < c_hi)
            cs = plsc.cumsum(m.astype(jnp.int32))
            pos = offv + cs - 1
            h = ((v - c_lo) << _JBITS) | jv
            plsc.store_scatter(hits, [pos], h, mask=m)
            return offv + plsc.all_reduce_population_count(m)

        offv = lax.fori_loop(0, _SEG // _L, scan_step, offv)

        @pl.when(s + 1 < _B // _SEG)
        def _():
            pltpu.async_copy(
                idx_hbm.at[pl.ds((s + 1) * _SEG, _SEG)], idxv, semi)

        return offv

    pltpu.async_copy(idx_hbm.at[pl.ds(0, _SEG)], idxv, semi)
    offv = lax.fori_loop(0, _B // _SEG, seg_step,
                         jnp.zeros((_L,), jnp.int32))
    neg = jnp.full((_L,), -1, jnp.int32)
    for q in range(_GB + 1):
        plsc.store_scatter(hits, [offv + iota + q * _L], neg)

    z = jnp.zeros((_L,), jnp.float32)
    for q in range(_GB):
        for r in range(_L):
            for cb in range(2 * _D // _L):
                stage[q, r, pl.ds(cb * _L, _L)] = z

    def zero_loop(r, _):
        pltpu.sync_copy(stage.at[0], shared.at[pl.ds(sid * (_SROWS // _NS)
                                                     + r * _L, _L)])
        return _

    lax.fori_loop(0, _SROWS // _NS // _L, zero_loop, 0)
    pltpu.sync_copy(stage.at[0, pl.ds(0, 1)],
                    shared.at[pl.ds(sid * (_SROWS // _NS)
                                    + (_SROWS // _NS // _L) * _L, 1)])
    plsc.subcore_barrier()

    def src(k):
        return tbl_hbm.at[:, pl.ds(pl.multiple_of((start + k) * _CHW, _CHW),
                                   _CHW)]

    pltpu.async_copy(src(0), bufa, sema)
    pltpu.async_copy(src(1), bufb, semb)

    for q in range(_GB):
        jbuf[q, pl.ds(0, _L)] = _HB + iota
        pltpu.async_copy(stage.at[q], shared.at[jbuf.at[q]], semsc,
                         add=True)

    def process(k, buf, sem):
        pltpu.make_async_copy(src(k), buf, sem).wait()
        h_lo = (k * _CHW) << _JBITS
        h_hi = ((k + 1) * _CHW) << _JBITS

        def cond(g):
            return ((g < _B // _L)
                    & jnp.any(hits[pl.ds(g * _L, _L)] >= 0))

        def wbody(g):
            for gg in range(_GB):
                h = hits[pl.ds((g + gg) * _L, _L)]
                m = (h >= h_lo) & (h < h_hi)
                pltpu.make_async_copy(stage.at[gg],
                                      shared.at[jbuf.at[gg]], semsc).wait()
                jh = jnp.where(m, (h & (_B - 1)) >> 1, _HB + iota)
                jbuf[gg, pl.ds(0, _L)] = jh

                @pl.when(jnp.any(m))
                def _():
                    local = jnp.where(m, (h >> _JBITS) - k * _CHW, 0)
                    half = ((h & (_B - 1)) & 1) * _D
                    for d in range(_D):
                        dv = jnp.full((_L,), d, jnp.int32)
                        vals = plsc.load_gather(buf, [dv, local])
                        plsc.store_scatter(stage.at[gg], [iota, dv + half],
                                           vals)
                        plsc.store_scatter(stage.at[gg],
                                           [iota, dv + (_D - half)],
                                           jnp.zeros((_L,), jnp.float32))

                pltpu.async_copy(stage.at[gg], shared.at[jbuf.at[gg]],
                                 semsc, add=True)

            return g + _GB

        lax.while_loop(cond, wbody, 0)

        @pl.when(k + 2 < nck)
        def _():
            pltpu.async_copy(src(k + 2), buf, sem)

    def chunk_body(k, _):
        @pl.when(k % 2 == 0)
        def _():
            process(k, bufa, sema)

        @pl.when(k % 2 == 1)
        def _():
            process(k, bufb, semb)

        return 0

    lax.fori_loop(0, nck, chunk_body, 0)
    for q in range(_GB):
        pltpu.make_async_copy(stage.at[q], shared.at[jbuf.at[q]],
                              semsc).wait()
    plsc.subcore_barrier()

    @pl.when(sid == 0)
    def _():
        pltpu.sync_copy(shared.at[pl.ds(0, _HB)], out01.at[cid])

    plsc.subcore_barrier()


def _gather_body(users_hbm, movies_hbm, ut_hbm, mt_hbm, gu, gm,
                 idxv, hits, bufa, bufb, stage, jbuf, shared,
                 sema, semb, semi, semsc):
    cid = lax.axis_index("c")
    sid = lax.axis_index("s")
    _stream_table(users_hbm, ut_hbm, gu, cid * _U_CK_SC, _U_CK_SC,
                  idxv, hits, bufa, bufb, stage, jbuf, shared,
                  sema, semb, semi, semsc, cid, sid)
    _stream_table(movies_hbm, mt_hbm, gm, cid * _M_CK_SC, _M_CK_SC,
                  idxv, hits, bufa, bufb, stage, jbuf, shared,
                  sema, semb, semi, semsc, cid, sid)


def _sc_gather(users, movies, ut_t, mt_t):
    mesh = plsc.VectorSubcoreMesh(core_axis_name="c", subcore_axis_name="s")
    out = jax.ShapeDtypeStruct((_NC, _HB, 2 * _D), jnp.float32)
    return pl.kernel(
        _gather_body,
        mesh=mesh,
        compiler_params=pltpu.CompilerParams(needs_layout_passes=False),
        out_type=[out, out],
        scratch_types=[
            pltpu.VMEM((_SEG,), jnp.int32),
            pltpu.VMEM((_B + 6 * _L,), jnp.int32),
            pltpu.VMEM((_D, _CHW), jnp.float32),
            pltpu.VMEM((_D, _CHW), jnp.float32),
            pltpu.VMEM((_GB, _L, 2 * _D), jnp.float32),
            pltpu.VMEM((_GB, _L), jnp.int32),
            pltpu.VMEM_SHARED((_SROWS, 2 * _D), jnp.float32),
            pltpu.SemaphoreType.DMA,
            pltpu.SemaphoreType.DMA,
            pltpu.SemaphoreType.DMA,
            pltpu.SemaphoreType.DMA,
        ],
    )(users, movies, ut_t, mt_t)


_BS = 2048


def _mlp_body(gu_ref, gm_ref, u_ref, m_ref,
              utail_ref, mtail_ref, w1_ref, b1_ref, w2_ref, b2_ref,
              w3_ref, b3_ref, w4_ref, b4_ref, out_ref):
    u = u_ref[...]
    mv = m_ref[...]
    xu = jnp.where(u < _U_SPLIT, gu_ref[0], gu_ref[1])
    iota_u = lax.broadcasted_iota(jnp.int32, (_BS, _D), 1)
    ohu = (u - _U_MAIN == iota_u).astype(jnp.float32)
    xu = jnp.where(u >= _U_MAIN,
                   jnp.dot(ohu, utail_ref[...],
                           preferred_element_type=jnp.float32), xu)
    xm = jnp.where(mv < _M_SPLIT, gm_ref[0], gm_ref[1])
    iota_m = lax.broadcasted_iota(jnp.int32, (_BS, _NM - _M_MAIN), 1)
    ohm = (mv - _M_MAIN == iota_m).astype(jnp.float32)
    xm = jnp.where(mv >= _M_MAIN,
                   jnp.dot(ohm, mtail_ref[...],
                           preferred_element_type=jnp.float32), xm)
    h = jnp.maximum(
        jnp.dot(xu, w1_ref[0:_D, :], preferred_element_type=jnp.float32)
        + jnp.dot(xm, w1_ref[_D:2 * _D, :],
                  preferred_element_type=jnp.float32)
        + b1_ref[...], 0.0)
    h = jnp.maximum(
        jnp.dot(h, w2_ref[...], preferred_element_type=jnp.float32)
        + b2_ref[...], 0.0)
    h = jnp.maximum(
        jnp.dot(h, w3_ref[...], preferred_element_type=jnp.float32)
        + b3_ref[...], 0.0)
    logit = jnp.sum(h * w4_ref[...], axis=1) + b4_ref[0, 0]
    out_ref[...] = 1.0 / (1.0 + jnp.exp(-logit))


def _tc_mlp(gu, gm, users2, movies2, utail, mtail,
            W1, b1, W2, b2, W3, b3, W4, b4):
    grid = (_B // _BS,)
    full = lambda shape: pl.BlockSpec(shape, lambda i: (0,) * len(shape))
    return pl.pallas_call(
        _mlp_body,
        grid=grid,
        in_specs=[
            pl.BlockSpec((_NC, _BS, _D), lambda i: (0, i, 0)),
            pl.BlockSpec((_NC, _BS, _D), lambda i: (0, i, 0)),
            pl.BlockSpec((_BS, 1), lambda i: (i, 0)),
            pl.BlockSpec((_BS, 1), lambda i: (i, 0)),
            full((_NU - _U_MAIN, _D)), full((_NM - _M_MAIN, _D)),
            full((2 * _D, 256)), full((1, 256)),
            full((256, 128)), full((1, 128)),
            full((128, _D)), full((1, _D)),
            full((1, _D)), full((1, 1)),
        ],
        out_specs=pl.BlockSpec((_BS,), lambda i: (i,)),
        out_shape=jax.ShapeDtypeStruct((_B,), jnp.float32),
    )(gu, gm, users2, movies2, utail, mtail,
      W1, b1.reshape(1, 256), W2, b2.reshape(1, 128),
      W3, b3.reshape(1, _D), W4.reshape(1, _D), b4.reshape(1, 1))


def kernel(users, movies, user_table, movie_table,
           W1, b1, W2, b2, W3, b3, W4, b4):
    users = users.astype(jnp.int32)
    movies = movies.astype(jnp.int32)
    gu, gm = _sc_gather(users, movies, user_table.T, movie_table.T)
    gu = gu.reshape(_NC, _B, _D)
    gm = gm.reshape(_NC, _B, _D)
    return _tc_mlp(gu, gm,
                   users.reshape(_B, 1), movies.reshape(_B, 1),
                   user_table[_U_MAIN:], movie_table[_M_MAIN:],
                   W1, b1, W2, b2, W3, b3, W4, b4)


# speedup vs baseline: 2.1439x; 2.1439x over previous
import jax
import jax.numpy as jnp
from jax import lax
from jax.experimental import pallas as pl
from jax.experimental.pallas import tpu as pltpu, tpu_sc as plsc

_B = 16384
_D = 64

_NC, _NS = 2, 16
_NW = _NC * _NS
_BPW = _B // _NW
_LANES = 16


def _gather_body(users_hbm, movies_hbm, ut2_hbm, mt2_hbm, outu_hbm, outm_hbm,
                 idx, half, rows, sem):
    wid = lax.axis_index("s") * _NC + lax.axis_index("c")
    base = wid * _BPW

    def one_table(src_idx_hbm, table_hbm, out_hbm):
        pltpu.sync_copy(src_idx_hbm.at[pl.ds(base, _BPW)], idx)
        def halve(v, _):
            half[pl.ds(v * _LANES, _LANES)] = (
                idx[pl.ds(v * _LANES, _LANES)] >> 1)
            return 0
        lax.fori_loop(0, _BPW // _LANES, halve, 0, unroll=8)
        pltpu.async_copy(table_hbm.at[half], rows, sem).wait()
        pltpu.sync_copy(rows, out_hbm.at[pl.ds(base, _BPW)])

    one_table(users_hbm, ut2_hbm, outu_hbm)
    one_table(movies_hbm, mt2_hbm, outm_hbm)


def _sc_gather(users, movies, ut2, mt2):
    mesh = plsc.VectorSubcoreMesh(core_axis_name="c", subcore_axis_name="s")
    return pl.kernel(
        _gather_body,
        mesh=mesh,
        out_type=[jax.ShapeDtypeStruct((_B, 2 * _D), jnp.float32),
                  jax.ShapeDtypeStruct((_B, 2 * _D), jnp.float32)],
        scratch_types=[
            pltpu.VMEM((_BPW,), jnp.int32),
            pltpu.VMEM((_BPW,), jnp.int32),
            pltpu.VMEM((_BPW, 2 * _D), jnp.float32),
            pltpu.SemaphoreType.DMA,
        ],
    )(users, movies, ut2, mt2)


_BS = 2048


def _mlp_body(gu_ref, gm_ref, pu_ref, pm_ref, w1_ref, b1_ref, w2_ref, b2_ref,
              w3_ref, b3_ref, w4_ref, b4_ref, out_ref):
    pu = (pu_ref[...] & 1) == 1
    pm = (pm_ref[...] & 1) == 1
    xu = jnp.where(pu, gu_ref[:, _D:], gu_ref[:, :_D])
    xm = jnp.where(pm, gm_ref[:, _D:], gm_ref[:, :_D])
    h = jnp.maximum(
        jnp.dot(xu, w1_ref[0:_D, :], preferred_element_type=jnp.float32)
        + jnp.dot(xm, w1_ref[_D:2 * _D, :],
                  preferred_element_type=jnp.float32)
        + b1_ref[...], 0.0)
    h = jnp.maximum(
        jnp.dot(h, w2_ref[...], preferred_element_type=jnp.float32)
        + b2_ref[...], 0.0)
    h = jnp.maximum(
        jnp.dot(h, w3_ref[...], preferred_element_type=jnp.float32)
        + b3_ref[...], 0.0)
    logit = jnp.sum(h * w4_ref[...], axis=1) + b4_ref[0, 0]
    out_ref[...] = 1.0 / (1.0 + jnp.exp(-logit))


def _tc_mlp(gu, gm, users2d, movies2d, W1, b1, W2, b2, W3, b3, W4, b4):
    grid = (_B // _BS,)
    full = lambda shape: pl.BlockSpec(shape, lambda i: (0,) * len(shape))
    return pl.pallas_call(
        _mlp_body,
        grid=grid,
        in_specs=[
            pl.BlockSpec((_BS, 2 * _D), lambda i: (i, 0)),
            pl.BlockSpec((_BS, 2 * _D), lambda i: (i, 0)),
            pl.BlockSpec((_BS, 1), lambda i: (i, 0)),
            pl.BlockSpec((_BS, 1), lambda i: (i, 0)),
            full((2 * _D, 256)), full((1, 256)),
            full((256, 128)), full((1, 128)),
            full((128, _D)), full((1, _D)),
            full((1, _D)), full((1, 1)),
        ],
        out_specs=pl.BlockSpec((_BS,), lambda i: (i,)),
        out_shape=jax.ShapeDtypeStruct((_B,), jnp.float32),
    )(gu, gm, users2d, movies2d,
      W1, b1.reshape(1, 256), W2, b2.reshape(1, 128),
      W3, b3.reshape(1, _D), W4.reshape(1, _D), b4.reshape(1, 1))


def kernel(users, movies, user_table, movie_table,
           W1, b1, W2, b2, W3, b3, W4, b4):
    users = users.astype(jnp.int32)
    movies = movies.astype(jnp.int32)
    ut2 = user_table.reshape(-1, 2 * _D)
    mt2 = movie_table.reshape(-1, 2 * _D)
    gu, gm = _sc_gather(users, movies, ut2, mt2)
    return _tc_mlp(gu, gm, users.reshape(_B, 1), movies.reshape(_B, 1),
                   W1, b1, W2, b2, W3, b3, W4, b4)
